# baseline (speedup 1.0000x reference)
import functools

import jax
import jax.numpy as jnp
from jax import lax
from jax.experimental import pallas as pl
from jax.experimental.pallas import tpu as pltpu
from jax.experimental.pallas import tpu_sc as plsc

_NUM_CORES = 2
_NUM_SUBCORES = 16
_NUM_WORKERS = _NUM_CORES * _NUM_SUBCORES
_LANES = 16


def _embed_bag_mean(text, table):
    B, BAG = text.shape
    _, D = table.shape
    assert B % _NUM_WORKERS == 0
    assert D % _LANES == 0
    bpw = B // _NUM_WORKERS
    nch = D // _LANES

    mesh = plsc.VectorSubcoreMesh(core_axis_name="c", subcore_axis_name="s")

    @functools.partial(
        pl.kernel,
        out_type=jax.ShapeDtypeStruct((B, D), jnp.float32),
        mesh=mesh,
        compiler_params=pltpu.CompilerParams(use_tc_tiling_on_sc=False),
        scratch_types=[
            pltpu.VMEM((bpw, BAG), jnp.int32),
            pltpu.VMEM((BAG, D), jnp.float32),
            pltpu.VMEM((bpw, D), jnp.float32),
            pltpu.SemaphoreType.DMA,
        ],
    )
    def k(text_hbm, table_hbm, out_hbm, idx_v, rows_v, out_v, sem):
        wid = lax.axis_index("s") * _NUM_CORES + lax.axis_index("c")
        base = wid * bpw
        pltpu.sync_copy(text_hbm.at[pl.ds(base, bpw)], idx_v)

        def row_body(i, carry):
            pltpu.async_copy(table_hbm.at[idx_v.at[i]], rows_v, sem).wait()

            def j_body(j, accs):
                return tuple(
                    accs[c] + rows_v[j, pl.ds(c * _LANES, _LANES)]
                    for c in range(nch)
                )

            accs = lax.fori_loop(
                0, BAG, j_body,
                tuple(jnp.zeros((_LANES,), jnp.float32) for _ in range(nch)),
            )
            scale = jnp.float32(1.0 / BAG)
            for c in range(nch):
                out_v[i, pl.ds(c * _LANES, _LANES)] = accs[c] * scale
            return carry

        lax.fori_loop(0, bpw, row_body, 0)
        pltpu.sync_copy(out_v, out_hbm.at[pl.ds(base, bpw)])

    return k(text, table)


def _mlp(x, W1, b1, W2, b2):
    B, D = x.shape
    C = W2.shape[0]
    BM = 512
    assert B % BM == 0

    def body(x_ref, w1_ref, b1_ref, w2_ref, b2_ref, o_ref):
        h = lax.dot_general(
            x_ref[...], w1_ref[...], (((1,), (1,)), ((), ())),
            preferred_element_type=jnp.float32,
        )
        h = jnp.maximum(h + b1_ref[...], 0.0)
        o_ref[...] = lax.dot_general(
            h, w2_ref[...], (((1,), (1,)), ((), ())),
            preferred_element_type=jnp.float32,
        ) + b2_ref[...]

    return pl.pallas_call(
        body,
        grid=(B // BM,),
        in_specs=[
            pl.BlockSpec((BM, D), lambda i: (i, 0)),
            pl.BlockSpec((D, D), lambda i: (0, 0)),
            pl.BlockSpec((1, D), lambda i: (0, 0)),
            pl.BlockSpec((C, D), lambda i: (0, 0)),
            pl.BlockSpec((1, C), lambda i: (0, 0)),
        ],
        out_specs=pl.BlockSpec((BM, C), lambda i: (i, 0)),
        out_shape=jax.ShapeDtypeStruct((B, C), jnp.float32),
    )(x, W1, b1.reshape(1, D), W2, b2.reshape(1, C))


def kernel(text, offsets, table, W1, b1, W2, b2):
    del offsets
    pooled = _embed_bag_mean(text, table)
    return _mlp(pooled, W1, b1, W2, b2)


# speedup vs baseline: 1.5575x; 1.5575x over previous
import functools

import jax
import jax.numpy as jnp
from jax import lax
from jax.experimental import pallas as pl
from jax.experimental.pallas import tpu as pltpu
from jax.experimental.pallas import tpu_sc as plsc

_NUM_CORES = 2
_NUM_SUBCORES = 16
_NUM_WORKERS = _NUM_CORES * _NUM_SUBCORES
_LANES = 16


def _detile_dup(tableT):
    D, V = tableT.shape
    BM = 4096
    grid = (V + BM - 1) // BM

    def body(t_ref, o_ref):
        x = lax.transpose(t_ref[...], (1, 0))
        o_ref[...] = jnp.concatenate([x, x], axis=1)

    return pl.pallas_call(
        body,
        grid=(grid,),
        in_specs=[pl.BlockSpec((D, BM), lambda i: (0, i))],
        out_specs=pl.BlockSpec((BM, 2 * D), lambda i: (i, 0)),
        out_shape=jax.ShapeDtypeStruct((V, 2 * D), jnp.float32),
    )(tableT)


def _embed_bag_mean(text2, table2, B, BAG, D):
    npair = B // 2
    assert npair % _NUM_WORKERS == 0
    ppw = npair // _NUM_WORKERS
    bpw = 2 * ppw
    nch = D // _LANES
    PB = 2 * BAG

    mesh = plsc.VectorSubcoreMesh(core_axis_name="c", subcore_axis_name="s")

    @functools.partial(
        pl.kernel,
        out_type=jax.ShapeDtypeStruct((B, D), jnp.float32),
        mesh=mesh,
        scratch_types=[
            pltpu.VMEM((ppw, PB), jnp.int32),
            pltpu.VMEM((PB, 2 * D), jnp.float32),
            pltpu.VMEM((PB, 2 * D), jnp.float32),
            pltpu.VMEM((bpw, D), jnp.float32),
            pltpu.SemaphoreType.DMA,
            pltpu.SemaphoreType.DMA,
        ],
    )
    def k(text_hbm, table_hbm, out_hbm, idx_v, rows0, rows1, out_v, sem0, sem1):
        wid = lax.axis_index("s") * _NUM_CORES + lax.axis_index("c")
        base = wid * ppw
        pltpu.sync_copy(text_hbm.at[pl.ds(base, ppw)], idx_v)

        scale = jnp.float32(1.0 / BAG)

        def accum(p, rows_v):
            def j_body(j, accs):
                new = []
                for h in range(2):
                    for c in range(nch):
                        new.append(
                            accs[h * nch + c]
                            + rows_v[h * BAG + j, pl.ds(c * _LANES, _LANES)]
                        )
                return tuple(new)

            accs = lax.fori_loop(
                0, BAG, j_body,
                tuple(jnp.zeros((_LANES,), jnp.float32) for _ in range(2 * nch)),
            )
            for h in range(2):
                for c in range(nch):
                    out_v[2 * p + h, pl.ds(c * _LANES, _LANES)] = (
                        accs[h * nch + c] * scale
                    )

        pltpu.async_copy(table_hbm.at[idx_v.at[0]], rows0, sem0)
        pltpu.async_copy(table_hbm.at[idx_v.at[1]], rows1, sem1)

        def pair_body(i, carry):
            p0 = 2 * i
            pltpu.make_async_copy(table_hbm.at[idx_v.at[p0]], rows0, sem0).wait()
            accum(p0, rows0)

            @pl.when(i < ppw // 2 - 1)
            def _():
                pltpu.async_copy(table_hbm.at[idx_v.at[p0 + 2]], rows0, sem0)

            p1 = p0 + 1
            pltpu.make_async_copy(table_hbm.at[idx_v.at[p1]], rows1, sem1).wait()
            accum(p1, rows1)

            @pl.when(i < ppw // 2 - 1)
            def _():
                pltpu.async_copy(table_hbm.at[idx_v.at[p1 + 2]], rows1, sem1)

            return carry

        lax.fori_loop(0, ppw // 2, pair_body, 0)
        pltpu.sync_copy(out_v, out_hbm.at[pl.ds(wid * bpw, bpw)])

    return k(text2, table2)


def _mlp(x, W1, b1, W2, b2):
    B, D = x.shape
    C = W2.shape[0]
    BM = 512
    assert B % BM == 0

    def body(x_ref, w1_ref, b1_ref, w2_ref, b2_ref, o_ref):
        h = lax.dot_general(
            x_ref[...], w1_ref[...], (((1,), (1,)), ((), ())),
            preferred_element_type=jnp.float32,
        )
        h = jnp.maximum(h + b1_ref[...], 0.0)
        o_ref[...] = lax.dot_general(
            h, w2_ref[...], (((1,), (1,)), ((), ())),
            preferred_element_type=jnp.float32,
        ) + b2_ref[...]

    return pl.pallas_call(
        body,
        grid=(B // BM,),
        in_specs=[
            pl.BlockSpec((BM, D), lambda i: (i, 0)),
            pl.BlockSpec((D, D), lambda i: (0, 0)),
            pl.BlockSpec((1, D), lambda i: (0, 0)),
            pl.BlockSpec((C, D), lambda i: (0, 0)),
            pl.BlockSpec((1, C), lambda i: (0, 0)),
        ],
        out_specs=pl.BlockSpec((BM, C), lambda i: (i, 0)),
        out_shape=jax.ShapeDtypeStruct((B, C), jnp.float32),
    )(x, W1, b1.reshape(1, D), W2, b2.reshape(1, C))


def kernel(text, offsets, table, W1, b1, W2, b2):
    del offsets
    B, BAG = text.shape
    D = table.shape[1]
    table2 = _detile_dup(table.T)
    text2 = text.reshape(B // 2, 2 * BAG)
    pooled = _embed_bag_mean(text2, table2, B, BAG, D)
    return _mlp(pooled, W1, b1, W2, b2)


# speedup vs baseline: 1.8334x; 1.1771x over previous
import functools

import jax
import jax.numpy as jnp
from jax import lax
from jax.experimental import pallas as pl
from jax.experimental.pallas import tpu as pltpu
from jax.experimental.pallas import tpu_sc as plsc

_NUM_CORES = 2
_NUM_SUBCORES = 16
_NUM_WORKERS = _NUM_CORES * _NUM_SUBCORES
_LANES = 16


def _detile_dup(tableT):
    D, V = tableT.shape
    BM = 8192
    grid = (V + BM - 1) // BM

    def body(t_ref, o_ref):
        x = lax.transpose(t_ref[...], (1, 0))
        o_ref[:, 0:D] = x
        o_ref[:, D:2 * D] = x

    return pl.pallas_call(
        body,
        grid=(grid,),
        in_specs=[pl.BlockSpec((D, BM), lambda i: (0, i))],
        out_specs=pl.BlockSpec((BM, 2 * D), lambda i: (i, 0)),
        out_shape=jax.ShapeDtypeStruct((V, 2 * D), jnp.float32),
    )(tableT)


def _embed_bag_mean(text2, table2, B, BAG, D):
    npair = B // 2
    assert npair % _NUM_WORKERS == 0
    ppw = npair // _NUM_WORKERS
    bpw = 2 * ppw
    nch = D // _LANES
    PB = 2 * BAG

    mesh = plsc.VectorSubcoreMesh(core_axis_name="c", subcore_axis_name="s")

    @functools.partial(
        pl.kernel,
        out_type=jax.ShapeDtypeStruct((B, D), jnp.float32),
        mesh=mesh,
        scratch_types=[
            pltpu.VMEM((ppw, PB), jnp.int32),
            pltpu.VMEM((PB, 2 * D), jnp.float32),
            pltpu.VMEM((PB, 2 * D), jnp.float32),
            pltpu.VMEM((bpw, D), jnp.float32),
            pltpu.SemaphoreType.DMA,
            pltpu.SemaphoreType.DMA,
        ],
    )
    def k(text_hbm, table_hbm, out_hbm, idx_v, rows0, rows1, out_v, sem0, sem1):
        wid = lax.axis_index("s") * _NUM_CORES + lax.axis_index("c")
        base = wid * ppw
        pltpu.sync_copy(text_hbm.at[pl.ds(base, ppw)], idx_v)

        scale = jnp.float32(1.0 / BAG)

        def accum(p, rows_v):
            def j_body(j, accs):
                new = []
                for h in range(2):
                    for c in range(nch):
                        new.append(
                            accs[h * nch + c]
                            + rows_v[h * BAG + j, pl.ds(c * _LANES, _LANES)]
                        )
                return tuple(new)

            accs = lax.fori_loop(
                0, BAG, j_body,
                tuple(jnp.zeros((_LANES,), jnp.float32) for _ in range(2 * nch)),
            )
            for h in range(2):
                for c in range(nch):
                    out_v[2 * p + h, pl.ds(c * _LANES, _LANES)] = (
                        accs[h * nch + c] * scale
                    )

        pltpu.async_copy(table_hbm.at[idx_v.at[0]], rows0, sem0)
        pltpu.async_copy(table_hbm.at[idx_v.at[1]], rows1, sem1)

        def pair_body(i, carry):
            p0 = 2 * i
            pltpu.make_async_copy(table_hbm.at[idx_v.at[p0]], rows0, sem0).wait()
            accum(p0, rows0)

            @pl.when(i < ppw // 2 - 1)
            def _():
                pltpu.async_copy(table_hbm.at[idx_v.at[p0 + 2]], rows0, sem0)

            p1 = p0 + 1
            pltpu.make_async_copy(table_hbm.at[idx_v.at[p1]], rows1, sem1).wait()
            accum(p1, rows1)

            @pl.when(i < ppw // 2 - 1)
            def _():
                pltpu.async_copy(table_hbm.at[idx_v.at[p1 + 2]], rows1, sem1)

            return carry

        lax.fori_loop(0, ppw // 2, pair_body, 0)
        pltpu.sync_copy(out_v, out_hbm.at[pl.ds(wid * bpw, bpw)])

    return k(text2, table2)


def _mlp(x, W1, b1, W2, b2):
    B, D = x.shape
    C = W2.shape[0]
    BM = 512
    assert B % BM == 0

    def body(x_ref, w1_ref, b1_ref, w2_ref, b2_ref, o_ref):
        h = lax.dot_general(
            x_ref[...], w1_ref[...], (((1,), (1,)), ((), ())),
            preferred_element_type=jnp.float32,
        )
        h = jnp.maximum(h + b1_ref[...], 0.0)
        o_ref[...] = lax.dot_general(
            h, w2_ref[...], (((1,), (1,)), ((), ())),
            preferred_element_type=jnp.float32,
        ) + b2_ref[...]

    return pl.pallas_call(
        body,
        grid=(B // BM,),
        in_specs=[
            pl.BlockSpec((BM, D), lambda i: (i, 0)),
            pl.BlockSpec((D, D), lambda i: (0, 0)),
            pl.BlockSpec((1, D), lambda i: (0, 0)),
            pl.BlockSpec((C, D), lambda i: (0, 0)),
            pl.BlockSpec((1, C), lambda i: (0, 0)),
        ],
        out_specs=pl.BlockSpec((BM, C), lambda i: (i, 0)),
        out_shape=jax.ShapeDtypeStruct((B, C), jnp.float32),
    )(x, W1, b1.reshape(1, D), W2, b2.reshape(1, C))


def kernel(text, offsets, table, W1, b1, W2, b2):
    del offsets
    B, BAG = text.shape
    D = table.shape[1]
    table2 = _detile_dup(table.T)
    text2 = text.reshape(B // 2, 2 * BAG)
    pooled = _embed_bag_mean(text2, table2, B, BAG, D)
    return _mlp(pooled, W1, b1, W2, b2)
